# Initial kernel scaffold; baseline (speedup 1.0000x reference)
import functools

import jax
import jax.numpy as jnp
from jax import lax
from jax.experimental import pallas as pl
from jax.experimental.pallas import tpu as pltpu
from jax.experimental.pallas import tpu_sc as plsc

_N = 10000
_NP = 10240
_C = 128
_E = 320000
_PE = 327680
_EPT = _PE // 32
_CHUNK = 128
_NCHUNK = _EPT // _CHUNK
_RPT = _NP // 16
_TILE = 256
_GRID = _NP // _TILE

_MESH = plsc.VectorSubcoreMesh(
    core_axis_name="c", subcore_axis_name="s", num_cores=2, num_subcores=16
)


@functools.partial(
    pl.kernel,
    out_type=jax.ShapeDtypeStruct((2, _NP, 16), jnp.float32),
    mesh=_MESH,
    scratch_types=[
        pltpu.VMEM((_NCHUNK, _CHUNK), jnp.int32),
        pltpu.VMEM((_CHUNK, 16), jnp.float32),
        pltpu.VMEM((_RPT, 16), jnp.float32),
        pltpu.VMEM_SHARED((_NP, 16), jnp.float32),
    ],
)
def _deg_kernel(dst3, degp, idx_d, ones_v, buf, acc):
    c = lax.axis_index("c")
    s = lax.axis_index("s")
    wid = s * 2 + c
    one16 = jnp.ones((16,), jnp.float32)
    zero16 = jnp.zeros((16,), jnp.float32)

    def fill_ones(i, carry):
        ones_v[i] = one16
        return carry

    lax.fori_loop(0, _CHUNK, fill_ones, 0)

    def fill_zero(i, carry):
        buf[i] = zero16
        return carry

    lax.fori_loop(0, _RPT, fill_zero, 0)
    base = s * _RPT
    pltpu.sync_copy(buf, acc.at[pl.ds(base, _RPT)])
    pltpu.sync_copy(dst3.at[wid], idx_d)
    plsc.subcore_barrier()

    def body(i, carry):
        pltpu.sync_copy(ones_v, acc.at[idx_d.at[i]], add=True)
        return carry

    lax.fori_loop(0, _NCHUNK, body, 0)
    plsc.subcore_barrier()
    pltpu.sync_copy(acc.at[pl.ds(base, _RPT)], buf)
    pltpu.sync_copy(buf, degp.at[c, pl.ds(base, _RPT)])


@functools.partial(
    pl.kernel,
    out_type=jax.ShapeDtypeStruct((2, _NP, _C), jnp.float32),
    mesh=_MESH,
    scratch_types=[
        pltpu.VMEM((_NCHUNK, _CHUNK), jnp.int32),
        pltpu.VMEM((_NCHUNK, _CHUNK), jnp.int32),
        pltpu.VMEM((_CHUNK, _C), jnp.float32),
        pltpu.VMEM_SHARED((_NP, _C), jnp.float32),
        pltpu.SemaphoreType.DMA,
    ],
)
def _prop_kernel(m2, src3, dst3, p_out, idx_s, idx_d, rows, acc, sem):
    c = lax.axis_index("c")
    s = lax.axis_index("s")
    wid = s * 2 + c
    zero16 = jnp.zeros((16,), jnp.float32)

    def fill_zero(i, carry):
        for j in range(_C // 16):
            rows[i, pl.ds(j * 16, 16)] = zero16
        return carry

    lax.fori_loop(0, _CHUNK, fill_zero, 0)
    base = s * _RPT
    for k in range(_RPT // _CHUNK):
        pltpu.sync_copy(rows, acc.at[pl.ds(base + k * _CHUNK, _CHUNK)])
    pltpu.sync_copy(src3.at[wid], idx_s)
    pltpu.sync_copy(dst3.at[wid], idx_d)
    plsc.subcore_barrier()

    def body(i, carry):
        pltpu.async_copy(m2.at[idx_s.at[i]], rows, sem).wait()
        pltpu.sync_copy(rows, acc.at[idx_d.at[i]], add=True)
        return carry

    lax.fori_loop(0, _NCHUNK, body, 0)
    plsc.subcore_barrier()
    for k in range(_RPT // _CHUNK):
        pltpu.sync_copy(acc.at[pl.ds(base + k * _CHUNK, _CHUNK)], rows)
        pltpu.sync_copy(rows, p_out.at[c, pl.ds(base + k * _CHUNK, _CHUNK)])


def _selu(z):
    alpha = 1.6732632423543772
    scale = 1.0507009873554805
    return scale * jnp.where(z > 0, z, alpha * jnp.expm1(z))


def _silu(z):
    return z / (1.0 + jnp.exp(-z))


def _logsig(z):
    return jnp.where(z >= 0, -jnp.log1p(jnp.exp(-z)), z - jnp.log1p(jnp.exp(z)))


def _pre_body(x_ref, degp_ref, w_ref, m_ref, dinv_ref):
    d = degp_ref[...]
    deg = d[0, :, 0] + d[1, :, 0] + 1.0
    dinv = lax.rsqrt(deg)
    m = jnp.dot(x_ref[...], w_ref[...], preferred_element_type=jnp.float32)
    m_ref[...] = m * dinv[:, None]
    dinv_ref[0, 0, :] = dinv


def _pre_call(x_pad, degp, w0):
    return pl.pallas_call(
        _pre_body,
        grid=(_GRID,),
        in_specs=[
            pl.BlockSpec((_TILE, _C), lambda i: (i, 0)),
            pl.BlockSpec((2, _TILE, 16), lambda i: (0, i, 0)),
            pl.BlockSpec((_C, _C), lambda i: (0, 0)),
        ],
        out_specs=[
            pl.BlockSpec((_TILE, _C), lambda i: (i, 0)),
            pl.BlockSpec((1, 1, _TILE), lambda i: (i, 0, 0)),
        ],
        out_shape=[
            jax.ShapeDtypeStruct((_NP, _C), jnp.float32),
            jax.ShapeDtypeStruct((_GRID, 1, _TILE), jnp.float32),
        ],
    )(x_pad, degp, w0)


def _layer_body(act, p_ref, m_ref, dinv_ref, b_ref, w_ref, out_ref):
    dinv = dinv_ref[0, 0, :]
    p = p_ref[...]
    z = (p[0] + p[1] + m_ref[...]) * dinv[:, None] + b_ref[...]
    h = act(z)
    hw = jnp.dot(h, w_ref[...], preferred_element_type=jnp.float32)
    out_ref[...] = hw * dinv[:, None]


def _layer_call(act, p, m, dinv3, b2d, w):
    return pl.pallas_call(
        functools.partial(_layer_body, act),
        grid=(_GRID,),
        in_specs=[
            pl.BlockSpec((2, _TILE, _C), lambda i: (0, i, 0)),
            pl.BlockSpec((_TILE, _C), lambda i: (i, 0)),
            pl.BlockSpec((1, 1, _TILE), lambda i: (i, 0, 0)),
            pl.BlockSpec((1, _C), lambda i: (0, 0)),
            pl.BlockSpec((_C, _C), lambda i: (0, 0)),
        ],
        out_specs=pl.BlockSpec((_TILE, _C), lambda i: (i, 0)),
        out_shape=jax.ShapeDtypeStruct((_NP, _C), jnp.float32),
    )(p, m, dinv3, b2d, w)


def _post_body(p_ref, m_ref, dinv_ref, b_ref, out_ref):
    dinv = dinv_ref[0, 0, :]
    p = p_ref[...]
    out_ref[...] = (p[0] + p[1] + m_ref[...]) * dinv[:, None] + b_ref[...]


def _post_call(p, m, dinv3, b2d):
    return pl.pallas_call(
        _post_body,
        grid=(_GRID,),
        in_specs=[
            pl.BlockSpec((2, _TILE, _C), lambda i: (0, i, 0)),
            pl.BlockSpec((_TILE, _C), lambda i: (i, 0)),
            pl.BlockSpec((1, 1, _TILE), lambda i: (i, 0, 0)),
            pl.BlockSpec((1, _C), lambda i: (0, 0)),
        ],
        out_specs=pl.BlockSpec((_TILE, _C), lambda i: (i, 0)),
        out_shape=jax.ShapeDtypeStruct((_NP, _C), jnp.float32),
    )(p, m, dinv3, b2d)


def kernel(x, edge_index, W0, b0, W1, b1, W2, b2, Wmu, bmu, Wlv, blv):
    src = edge_index[0].astype(jnp.int32)
    dst = edge_index[1].astype(jnp.int32)
    npad_e = _PE - _E
    spread = (jnp.arange(npad_e, dtype=jnp.int32) % (_NP - _N)) + _N
    src3 = jnp.concatenate([src, spread]).reshape(32, _NCHUNK, _CHUNK)
    dst3 = jnp.concatenate([dst, spread]).reshape(32, _NCHUNK, _CHUNK)
    x_pad = jnp.pad(x, ((0, _NP - _N), (0, 0)))

    degp = _deg_kernel(dst3)
    m0, dinv3 = _pre_call(x_pad, degp, W0)
    p0 = _prop_kernel(m0, src3, dst3)
    m1 = _layer_call(_selu, p0, m0, dinv3, b0.reshape(1, _C), W1)
    p1 = _prop_kernel(m1, src3, dst3)
    m2 = _layer_call(_silu, p1, m1, dinv3, b1.reshape(1, _C), W2)
    p2 = _prop_kernel(m2, src3, dst3)
    wml = jnp.concatenate([Wmu, Wlv], axis=1)
    m3 = _layer_call(_logsig, p2, m2, dinv3, b2.reshape(1, _C), wml)
    p3 = _prop_kernel(m3, src3, dst3)
    bml = jnp.concatenate([bmu, blv]).reshape(1, _C)
    out = _post_call(p3, m3, dinv3, bml)
    return out[:_N, :64], out[:_N, 64:]


# speedup vs baseline: 15.1823x; 15.1823x over previous
import functools

import jax
import jax.numpy as jnp
from jax import lax
from jax.experimental import pallas as pl
from jax.experimental.pallas import tpu as pltpu
from jax.experimental.pallas import tpu_sc as plsc

_N = 10000
_NP = 10240
_C = 128
_E = 320000
_PE = 327680
_EPT = _PE // 32
_CHUNK = 128
_NCHUNK = _EPT // _CHUNK
_RPT = _NP // 16
_TILE = 256
_GRID = _NP // _TILE


@functools.cache
def _sc_kernels():
    mesh = plsc.VectorSubcoreMesh(
        core_axis_name="c", subcore_axis_name="s", num_cores=2, num_subcores=16
    )
    prop = functools.partial(
        pl.kernel,
        out_type=jax.ShapeDtypeStruct((2, _NP, _C), jnp.float32),
        mesh=mesh,
        scratch_types=[
            pltpu.VMEM((_NCHUNK, _CHUNK), jnp.int32),
            pltpu.VMEM((_NCHUNK, _CHUNK), jnp.int32),
            pltpu.VMEM((_CHUNK,), jnp.int32),
            pltpu.VMEM((_CHUNK, _C), jnp.float32),
            pltpu.VMEM_SHARED((_NP, _C), jnp.float32),
            pltpu.SemaphoreType.DMA,
        ],
    )(_prop_body)
    return (prop,)


def _prop_kernel(m2, src3, dst3):
    return _sc_kernels()[0](m2, src3, dst3)


def _fill_iota(idx_cur, start):
    iota16 = jnp.arange(16, dtype=jnp.int32)
    for j in range(_CHUNK // 16):
        idx_cur[pl.ds(j * 16, 16)] = start + j * 16 + iota16


def _stage_idx(idx_cur, idx2d, i):
    for j in range(_CHUNK // 16):
        idx_cur[pl.ds(j * 16, 16)] = idx2d[i, pl.ds(j * 16, 16)]


def _prop_body(m2, src3, dst3, p_out, idx_s, idx_d, idx_cur, rows, acc, sem):
    c = lax.axis_index("c")
    s = lax.axis_index("s")
    wid = s * 2 + c
    base = s * _RPT
    zero16 = jnp.zeros((16,), jnp.float32)
    nck = _RPT // _CHUNK

    def fill_zero(i, carry):
        for j in range(_C // 16):
            rows[i, pl.ds(j * 16, 16)] = zero16
        return carry

    lax.fori_loop(0, _CHUNK, fill_zero, 0)
    for r in range(nck):
        _fill_iota(idx_cur, base + r * _CHUNK)
        pltpu.sync_copy(rows, acc.at[idx_cur])
    pltpu.sync_copy(src3.at[wid], idx_s)
    pltpu.sync_copy(dst3.at[wid], idx_d)
    plsc.subcore_barrier()

    def body(i, carry):
        _stage_idx(idx_cur, idx_s, i)
        pltpu.async_copy(m2.at[idx_cur], rows, sem).wait()
        _stage_idx(idx_cur, idx_d, i)
        pltpu.sync_copy(rows, acc.at[idx_cur], add=True)
        return carry

    lax.fori_loop(0, _NCHUNK, body, 0)
    plsc.subcore_barrier()
    for r in range(nck):
        _fill_iota(idx_cur, base + r * _CHUNK)
        pltpu.async_copy(acc.at[idx_cur], rows, sem).wait()
        pltpu.sync_copy(rows, p_out.at[c, pl.ds(base + r * _CHUNK, _CHUNK)])


def _selu(z):
    alpha = 1.6732632423543772
    scale = 1.0507009873554805
    return scale * jnp.where(z > 0, z, alpha * (jnp.exp(jnp.minimum(z, 0.0)) - 1.0))


def _silu(z):
    return z / (1.0 + jnp.exp(-z))


def _logsig(z):
    zn = -jnp.abs(z)
    return jnp.minimum(z, 0.0) - jnp.log(1.0 + jnp.exp(zn))


def _pre_body(x_ref, degp_ref, w_ref, m_ref, dinv_ref):
    d = degp_ref[...]
    deg = d[0, :, 0] + d[1, :, 0] + 1.0
    dinv = lax.rsqrt(deg)
    m = jnp.dot(x_ref[...], w_ref[...], preferred_element_type=jnp.float32)
    m_ref[...] = m * dinv[:, None]
    dinv_ref[0, 0, :] = dinv


def _pre_call(x_pad, degp, w0):
    return pl.pallas_call(
        _pre_body,
        grid=(_GRID,),
        in_specs=[
            pl.BlockSpec((_TILE, _C), lambda i: (i, 0)),
            pl.BlockSpec((2, _TILE, _C), lambda i: (0, i, 0)),
            pl.BlockSpec((_C, _C), lambda i: (0, 0)),
        ],
        out_specs=[
            pl.BlockSpec((_TILE, _C), lambda i: (i, 0)),
            pl.BlockSpec((1, 1, _TILE), lambda i: (i, 0, 0)),
        ],
        out_shape=[
            jax.ShapeDtypeStruct((_NP, _C), jnp.float32),
            jax.ShapeDtypeStruct((_GRID, 1, _TILE), jnp.float32),
        ],
    )(x_pad, degp, w0)


def _layer_body(act, p_ref, m_ref, dinv_ref, b_ref, w_ref, out_ref):
    dinv = dinv_ref[0, 0, :]
    p = p_ref[...]
    z = (p[0] + p[1] + m_ref[...]) * dinv[:, None] + b_ref[...]
    h = act(z)
    hw = jnp.dot(h, w_ref[...], preferred_element_type=jnp.float32)
    out_ref[...] = hw * dinv[:, None]


def _layer_call(act, p, m, dinv3, b2d, w):
    return pl.pallas_call(
        functools.partial(_layer_body, act),
        grid=(_GRID,),
        in_specs=[
            pl.BlockSpec((2, _TILE, _C), lambda i: (0, i, 0)),
            pl.BlockSpec((_TILE, _C), lambda i: (i, 0)),
            pl.BlockSpec((1, 1, _TILE), lambda i: (i, 0, 0)),
            pl.BlockSpec((1, _C), lambda i: (0, 0)),
            pl.BlockSpec((_C, _C), lambda i: (0, 0)),
        ],
        out_specs=pl.BlockSpec((_TILE, _C), lambda i: (i, 0)),
        out_shape=jax.ShapeDtypeStruct((_NP, _C), jnp.float32),
    )(p, m, dinv3, b2d, w)


def _post_body(p_ref, m_ref, dinv_ref, b_ref, out_ref):
    dinv = dinv_ref[0, 0, :]
    p = p_ref[...]
    out_ref[...] = (p[0] + p[1] + m_ref[...]) * dinv[:, None] + b_ref[...]


def _post_call(p, m, dinv3, b2d):
    return pl.pallas_call(
        _post_body,
        grid=(_GRID,),
        in_specs=[
            pl.BlockSpec((2, _TILE, _C), lambda i: (0, i, 0)),
            pl.BlockSpec((_TILE, _C), lambda i: (i, 0)),
            pl.BlockSpec((1, 1, _TILE), lambda i: (i, 0, 0)),
            pl.BlockSpec((1, _C), lambda i: (0, 0)),
        ],
        out_specs=pl.BlockSpec((_TILE, _C), lambda i: (i, 0)),
        out_shape=jax.ShapeDtypeStruct((_NP, _C), jnp.float32),
    )(p, m, dinv3, b2d)


def kernel(x, edge_index, W0, b0, W1, b1, W2, b2, Wmu, bmu, Wlv, blv):
    src = edge_index[0].astype(jnp.int32)
    dst = edge_index[1].astype(jnp.int32)
    npad_e = _PE - _E
    spread = (jnp.arange(npad_e, dtype=jnp.int32) % (_NP - _N)) + _N
    src3 = jnp.concatenate([src, spread]).reshape(32, _NCHUNK, _CHUNK)
    dst3 = jnp.concatenate([dst, spread]).reshape(32, _NCHUNK, _CHUNK)
    x_pad = jnp.pad(x, ((0, _NP - _N), (0, 0)))

    ones_m = jnp.ones((_NP, _C), jnp.float32)
    degp = _prop_kernel(ones_m, src3, dst3)
    m0, dinv3 = _pre_call(x_pad, degp, W0)
    p0 = _prop_kernel(m0, src3, dst3)
    m1 = _layer_call(_selu, p0, m0, dinv3, b0.reshape(1, _C), W1)
    p1 = _prop_kernel(m1, src3, dst3)
    m2 = _layer_call(_silu, p1, m1, dinv3, b1.reshape(1, _C), W2)
    p2 = _prop_kernel(m2, src3, dst3)
    wml = jnp.concatenate([Wmu, Wlv], axis=1)
    m3 = _layer_call(_logsig, p2, m2, dinv3, b2.reshape(1, _C), wml)
    p3 = _prop_kernel(m3, src3, dst3)
    bml = jnp.concatenate([bmu, blv]).reshape(1, _C)
    out = _post_call(p3, m3, dinv3, bml)
    return out[:_N, :64], out[:_N, 64:]


# speedup vs baseline: 22.8203x; 1.5031x over previous
import functools

import jax
import jax.numpy as jnp
from jax import lax
from jax.experimental import pallas as pl
from jax.experimental.pallas import tpu as pltpu
from jax.experimental.pallas import tpu_sc as plsc

_N = 10000
_NP = 10240
_C = 128
_E = 320000
_PE = 327680
_EPT = _PE // 32
_CHUNK = 128
_NCHUNK = _EPT // _CHUNK
_RPT = _NP // 16
_TILE = 256
_GRID = _NP // _TILE


@functools.cache
def _sc_kernels():
    mesh = plsc.VectorSubcoreMesh(
        core_axis_name="c", subcore_axis_name="s", num_cores=2, num_subcores=16
    )
    prop = functools.partial(
        pl.kernel,
        out_type=jax.ShapeDtypeStruct((2, _NP, _C), jnp.float32),
        mesh=mesh,
        scratch_types=[
            pltpu.VMEM((_NCHUNK, _CHUNK), jnp.int32),
            pltpu.VMEM((_CHUNK,), jnp.int32),
            pltpu.VMEM((_CHUNK,), jnp.int32),
            pltpu.VMEM((_CHUNK,), jnp.int32),
            pltpu.VMEM((_CHUNK, _C), jnp.float32),
            pltpu.VMEM((_CHUNK, _C), jnp.float32),
            pltpu.VMEM_SHARED((_NP, _C), jnp.float32),
            pltpu.SemaphoreType.DMA,
            pltpu.SemaphoreType.DMA,
        ],
    )(_prop_body)
    return (prop,)


def _prop_kernel(m2, ed3):
    return _sc_kernels()[0](m2, ed3)


def _fill_iota(idx_cur, start):
    iota16 = jnp.arange(16, dtype=jnp.int32)
    for j in range(_CHUNK // 16):
        idx_cur[pl.ds(j * 16, 16)] = start + j * 16 + iota16


def _stage_src(idx_cur, idxp, i):
    for j in range(_CHUNK // 16):
        v = idxp[i, pl.ds(j * 16, 16)]
        idx_cur[pl.ds(j * 16, 16)] = v & 16383


def _stage_dst(idx_cur, idxp, i):
    for j in range(_CHUNK // 16):
        v = idxp[i, pl.ds(j * 16, 16)]
        idx_cur[pl.ds(j * 16, 16)] = lax.shift_right_logical(v, 14)


def _prop_body(m2, ed3, p_out, idx_p, idx_g0, idx_g1, idx_sc,
               rows0, rows1, acc, sem0, sem1):
    c = lax.axis_index("c")
    s = lax.axis_index("s")
    wid = s * 2 + c
    base = s * _RPT
    zero16 = jnp.zeros((16,), jnp.float32)
    nck = _RPT // _CHUNK

    def fill_zero(i, carry):
        for j in range(_C // 16):
            rows0[i, pl.ds(j * 16, 16)] = zero16
        return carry

    lax.fori_loop(0, _CHUNK, fill_zero, 0)
    for r in range(nck):
        _fill_iota(idx_sc, base + r * _CHUNK)
        pltpu.sync_copy(rows0, acc.at[idx_sc])
    pltpu.sync_copy(ed3.at[wid], idx_p)
    plsc.subcore_barrier()

    _stage_src(idx_g0, idx_p, 0)
    pltpu.async_copy(m2.at[idx_g0], rows0, sem0)

    def body(t, carry):
        i0 = 2 * t
        _stage_src(idx_g1, idx_p, i0 + 1)
        pltpu.async_copy(m2.at[idx_g1], rows1, sem1)
        pltpu.make_async_copy(m2.at[idx_g0], rows0, sem0).wait()
        _stage_dst(idx_sc, idx_p, i0)
        pltpu.sync_copy(rows0, acc.at[idx_sc], add=True)
        i2 = lax.rem(i0 + 2, _NCHUNK)
        _stage_src(idx_g0, idx_p, i2)
        pltpu.async_copy(m2.at[idx_g0], rows0, sem0)
        pltpu.make_async_copy(m2.at[idx_g1], rows1, sem1).wait()
        _stage_dst(idx_sc, idx_p, i0 + 1)
        pltpu.sync_copy(rows1, acc.at[idx_sc], add=True)
        return carry

    lax.fori_loop(0, _NCHUNK // 2, body, 0)
    pltpu.make_async_copy(m2.at[idx_g0], rows0, sem0).wait()
    plsc.subcore_barrier()
    for r in range(nck):
        _fill_iota(idx_sc, base + r * _CHUNK)
        pltpu.async_copy(acc.at[idx_sc], rows0, sem0).wait()
        pltpu.sync_copy(rows0, p_out.at[c, pl.ds(base + r * _CHUNK, _CHUNK)])


def _selu(z):
    alpha = 1.6732632423543772
    scale = 1.0507009873554805
    return scale * jnp.where(z > 0, z, alpha * (jnp.exp(jnp.minimum(z, 0.0)) - 1.0))


def _silu(z):
    return z / (1.0 + jnp.exp(-z))


def _logsig(z):
    zn = -jnp.abs(z)
    return jnp.minimum(z, 0.0) - jnp.log(1.0 + jnp.exp(zn))


def _pre_body(x_ref, degp_ref, w_ref, m_ref, dinv_ref):
    d = degp_ref[...]
    deg = d[0, :, 0] + d[1, :, 0] + 1.0
    dinv = lax.rsqrt(deg)
    m = jnp.dot(x_ref[...], w_ref[...], preferred_element_type=jnp.float32)
    m_ref[...] = m * dinv[:, None]
    dinv_ref[0, 0, :] = dinv


def _pre_call(x_pad, degp, w0):
    return pl.pallas_call(
        _pre_body,
        grid=(_GRID,),
        in_specs=[
            pl.BlockSpec((_TILE, _C), lambda i: (i, 0)),
            pl.BlockSpec((2, _TILE, _C), lambda i: (0, i, 0)),
            pl.BlockSpec((_C, _C), lambda i: (0, 0)),
        ],
        out_specs=[
            pl.BlockSpec((_TILE, _C), lambda i: (i, 0)),
            pl.BlockSpec((1, 1, _TILE), lambda i: (i, 0, 0)),
        ],
        out_shape=[
            jax.ShapeDtypeStruct((_NP, _C), jnp.float32),
            jax.ShapeDtypeStruct((_GRID, 1, _TILE), jnp.float32),
        ],
    )(x_pad, degp, w0)


def _layer_body(act, p_ref, m_ref, dinv_ref, b_ref, w_ref, out_ref):
    dinv = dinv_ref[0, 0, :]
    p = p_ref[...]
    z = (p[0] + p[1] + m_ref[...]) * dinv[:, None] + b_ref[...]
    h = act(z)
    hw = jnp.dot(h, w_ref[...], preferred_element_type=jnp.float32)
    out_ref[...] = hw * dinv[:, None]


def _layer_call(act, p, m, dinv3, b2d, w):
    return pl.pallas_call(
        functools.partial(_layer_body, act),
        grid=(_GRID,),
        in_specs=[
            pl.BlockSpec((2, _TILE, _C), lambda i: (0, i, 0)),
            pl.BlockSpec((_TILE, _C), lambda i: (i, 0)),
            pl.BlockSpec((1, 1, _TILE), lambda i: (i, 0, 0)),
            pl.BlockSpec((1, _C), lambda i: (0, 0)),
            pl.BlockSpec((_C, _C), lambda i: (0, 0)),
        ],
        out_specs=pl.BlockSpec((_TILE, _C), lambda i: (i, 0)),
        out_shape=jax.ShapeDtypeStruct((_NP, _C), jnp.float32),
    )(p, m, dinv3, b2d, w)


def _post_body(p_ref, m_ref, dinv_ref, b_ref, out_ref):
    dinv = dinv_ref[0, 0, :]
    p = p_ref[...]
    out_ref[...] = (p[0] + p[1] + m_ref[...]) * dinv[:, None] + b_ref[...]


def _post_call(p, m, dinv3, b2d):
    return pl.pallas_call(
        _post_body,
        grid=(_GRID,),
        in_specs=[
            pl.BlockSpec((2, _TILE, _C), lambda i: (0, i, 0)),
            pl.BlockSpec((_TILE, _C), lambda i: (i, 0)),
            pl.BlockSpec((1, 1, _TILE), lambda i: (i, 0, 0)),
            pl.BlockSpec((1, _C), lambda i: (0, 0)),
        ],
        out_specs=pl.BlockSpec((_TILE, _C), lambda i: (i, 0)),
        out_shape=jax.ShapeDtypeStruct((_NP, _C), jnp.float32),
    )(p, m, dinv3, b2d)


def kernel(x, edge_index, W0, b0, W1, b1, W2, b2, Wmu, bmu, Wlv, blv):
    src = edge_index[0].astype(jnp.int32)
    dst = edge_index[1].astype(jnp.int32)
    npad_e = _PE - _E
    spread = (jnp.arange(npad_e, dtype=jnp.int32) % (_NP - _N)) + _N
    src_p = jnp.concatenate([src, spread])
    dst_p = jnp.concatenate([dst, spread])
    ed3 = (src_p + (dst_p << 14)).reshape(32, _NCHUNK, _CHUNK)
    x_pad = jnp.pad(x, ((0, _NP - _N), (0, 0)))

    ones_m = jnp.ones((_NP, _C), jnp.float32)
    degp = _prop_kernel(ones_m, ed3)
    m0, dinv3 = _pre_call(x_pad, degp, W0)
    p0 = _prop_kernel(m0, ed3)
    m1 = _layer_call(_selu, p0, m0, dinv3, b0.reshape(1, _C), W1)
    p1 = _prop_kernel(m1, ed3)
    m2 = _layer_call(_silu, p1, m1, dinv3, b1.reshape(1, _C), W2)
    p2 = _prop_kernel(m2, ed3)
    wml = jnp.concatenate([Wmu, Wlv], axis=1)
    m3 = _layer_call(_logsig, p2, m2, dinv3, b2.reshape(1, _C), wml)
    p3 = _prop_kernel(m3, ed3)
    bml = jnp.concatenate([bmu, blv]).reshape(1, _C)
    out = _post_call(p3, m3, dinv3, bml)
    return out[:_N, :64], out[:_N, 64:]
